# baseline (speedup 1.0000x reference)
import functools

import jax
import jax.numpy as jnp
from jax import lax
from jax.experimental import pallas as pl
from jax.experimental.pallas import tpu as pltpu
from jax.experimental.pallas import tpu_sc as plsc

N = 10000
E = 160000
D = 256
G = 512
H = 128
BN = 1000
NB = N // BN
F32 = jnp.float32


_EPT = E // 16
_CH = 80
_NCH = _EPT // _CH
_NCHP = 128
_R0 = 640
_WB = 80

_EPT_C = E // 32
_CHC = 125
_NCHC = _EPT_C // _CHC


def _tile_rows(s):
    r0 = s * _R0
    nch = jnp.minimum((N - r0) // _WB, _R0 // _WB)
    return r0, nch


def _zero_acc(s, wbuf, acc):
    r0, nch = _tile_rows(s)

    def zacc(j, carry):
        pltpu.sync_copy(wbuf, acc.at[pl.ds(r0 + j * _WB, _WB)])
        return carry

    lax.fori_loop(0, nch, zacc, 0)


def _write_acc(c, s, wbuf, acc, out_hbm):
    r0, nch = _tile_rows(s)

    def wb(j, carry):
        rj = r0 + j * _WB
        pltpu.sync_copy(acc.at[pl.ds(rj, _WB)], wbuf)
        pltpu.sync_copy(wbuf, out_hbm.at[pl.ds(c * N + rj, _WB)])
        return carry

    lax.fori_loop(0, nch, wb, 0)

def _zero_vmem_rows(buf, nrows):
    zero16 = jnp.zeros((16,), F32)

    def body(r, carry):
        for k in range(H // 16):
            buf[r, pl.ds(k * 16, 16)] = zero16
        return carry

    lax.fori_loop(0, nrows, body, 0)


@functools.cache
def _make_spmm_sc():
    mesh = plsc.VectorSubcoreMesh(
        core_axis_name="c", subcore_axis_name="s", num_cores=2, num_subcores=16
    )
    return pl.kernel(
        _spmm_sc_body,
        out_type=jax.ShapeDtypeStruct((2 * N, H), F32),
        mesh=mesh,
        scratch_types=[
            pltpu.VMEM((_EPT,), jnp.int32),
            pltpu.VMEM((_NCHP, _CH), jnp.int32),
            pltpu.VMEM((_CH, H), F32),
            pltpu.VMEM((_CH, H), F32),
            pltpu.VMEM_SHARED((N, H), F32),
            pltpu.SemaphoreType.DMA,
            pltpu.SemaphoreType.DMA,
            pltpu.SemaphoreType.DMA,
            pltpu.SemaphoreType.DMA,
        ],
    )


def _spmm_sc(h2, src2, dst):
    return _make_spmm_sc()(h2, src2, dst)


def _spmm_sc_body(
    h_hbm, src2_hbm, dst3_hbm, out_hbm,
    isrc, idst, rowsa, rowsb, acc, sema, semb, sca, scb
):
    c = lax.axis_index("c")
    s = lax.axis_index("s")

    _zero_vmem_rows(rowsa, _CH)
    _zero_acc(s, rowsa, acc)
    pltpu.sync_copy(src2_hbm.at[pl.ds(c * E + s * _EPT, _EPT)], isrc)
    pltpu.sync_copy(dst3_hbm.at[pl.ds(s * _NCHP, _NCHP)], idst)
    plsc.subcore_barrier()

    def src_at(ci):
        return isrc.at[pl.ds(ci * _CH, _CH)]

    pltpu.async_copy(h_hbm.at[src_at(0)], rowsa, sema)
    pltpu.make_async_copy(h_hbm.at[src_at(0)], rowsa, sema).wait()
    pltpu.async_copy(rowsa, acc.at[idst.at[0]], scb, add=True)
    pltpu.async_copy(h_hbm.at[src_at(1)], rowsb, semb)
    bufs = ((rowsa, sema, sca), (rowsb, semb, scb))

    def pair(k, carry):
        for b in range(2):
            crows, csem, csc = bufs[b]
            nrows, nsem, nsc = bufs[1 - b]
            ci = 2 * k + b + 1
            pltpu.make_async_copy(h_hbm.at[src_at(ci)], nrows, nsem).wait()
            pltpu.make_async_copy(crows, acc.at[idst.at[ci - 1]], nsc).wait()
            pltpu.async_copy(nrows, acc.at[idst.at[ci]], csc, add=True)
            pltpu.async_copy(h_hbm.at[src_at(ci + 1)], crows, csem)
        return carry

    lax.fori_loop(0, (_NCH - 3) // 2, pair, 0)
    pltpu.make_async_copy(h_hbm.at[src_at(_NCH - 2)], rowsb, semb).wait()
    pltpu.make_async_copy(rowsa, acc.at[idst.at[_NCH - 3]], scb).wait()
    pltpu.async_copy(h_hbm.at[src_at(_NCH - 1)], rowsa, sema)
    pltpu.async_copy(rowsb, acc.at[idst.at[_NCH - 2]], sca, add=True)
    pltpu.make_async_copy(h_hbm.at[src_at(_NCH - 1)], rowsa, sema).wait()
    pltpu.make_async_copy(rowsb, acc.at[idst.at[_NCH - 2]], sca).wait()
    pltpu.sync_copy(rowsa, acc.at[idst.at[_NCH - 1]], add=True)
    plsc.subcore_barrier()

    _write_acc(c, s, rowsa, acc, out_hbm)


@functools.cache
def _make_count_sc():
    mesh = plsc.VectorSubcoreMesh(
        core_axis_name="c", subcore_axis_name="s", num_cores=2, num_subcores=16
    )
    return pl.kernel(
        _count_sc_body,
        out_type=jax.ShapeDtypeStruct((2 * N, H), F32),
        mesh=mesh,
        scratch_types=[
            pltpu.VMEM((_NCHC, _CHC), jnp.int32),
            pltpu.VMEM((_CHC, H), F32),
            pltpu.VMEM((_WB, H), F32),
            pltpu.VMEM_SHARED((N, H), F32),
        ],
    )


def _count_sc(dst):
    return _make_count_sc()(dst)


def _count_sc_body(dstc_hbm, out_hbm, idst, ones, wbuf, acc):
    c = lax.axis_index("c")
    s = lax.axis_index("s")

    _zero_vmem_rows(wbuf, _WB)
    one16 = jnp.ones((16,), F32)

    def fill(r, carry):
        for k in range(H // 16):
            ones[r, pl.ds(k * 16, 16)] = one16
        return carry

    lax.fori_loop(0, _CHC, fill, 0)

    _zero_acc(s, wbuf, acc)
    pltpu.sync_copy(dstc_hbm.at[pl.ds((c * 16 + s) * _NCHC, _NCHC)], idst)
    plsc.subcore_barrier()

    def step(i, carry):
        pltpu.sync_copy(ones, acc.at[idst.at[i]], add=True)
        return carry

    lax.fori_loop(0, _NCHC, step, 0)
    plsc.subcore_barrier()

    _write_acc(c, s, wbuf, acc, out_hbm)


def _encode_body(x_ref, t_ref, out_ref):
    x = x_ref[...]
    acc = jnp.zeros((BN, H), F32)
    tb = t_ref[...]
    for f in range(9):
        col = x[:, f : f + 1]
        oh = (lax.broadcasted_iota(jnp.int32, (BN, 128), 1) == col).astype(F32)
        acc = acc + jnp.dot(
            oh, tb[f * 128 : (f + 1) * 128, :], preferred_element_type=F32, precision=lax.Precision.HIGHEST
        )
    out_ref[...] = acc


def _encode(x, tables):
    return pl.pallas_call(
        _encode_body,
        grid=(2, NB),
        in_specs=[
            pl.BlockSpec((BN, 9), lambda h, i: (i, 0)),
            pl.BlockSpec((9 * 128, H), lambda h, i: (0, h)),
        ],
        out_specs=pl.BlockSpec((BN, H), lambda h, i: (h * NB + i, 0)),
        out_shape=jax.ShapeDtypeStruct((2 * N, H), F32),
    )(x, tables)


def _layer_body(
    agg_a, agg_b, h_a, h_b, cnt_a, cnt_b, wl, wr, bias, g_ref, b_ref,
    out_ref, z_scr, s1_scr, s2_scr
):
    t = pl.program_id(1)

    @pl.when(t < NB)
    def _():
        inv = 1.0 / jnp.maximum(cnt_a[...] + cnt_b[...], 1.0)
        m = jnp.concatenate([agg_a[...] * inv, agg_b[...] * inv], axis=1)
        hp = jnp.concatenate([h_a[...], h_b[...]], axis=1)
        z = (
            jnp.dot(m, wl[...], preferred_element_type=F32)
            + jnp.dot(hp, wr[...], preferred_element_type=F32)
            + bias[0]
        )
        z_scr[t] = z
        cs = jnp.sum(z, axis=0, keepdims=True)

        @pl.when(t == 0)
        def _():
            s1_scr[...] = cs

        @pl.when(t > 0)
        def _():
            s1_scr[...] += cs

    @pl.when((t >= NB) & (t < 2 * NB))
    def _():
        mean = s1_scr[...] * (1.0 / N)
        dz = z_scr[t - NB] - mean
        cv = jnp.sum(dz * dz, axis=0, keepdims=True)

        @pl.when(t == NB)
        def _():
            s2_scr[...] = cv

        @pl.when(t > NB)
        def _():
            s2_scr[...] += cv

    @pl.when(t >= 2 * NB)
    def _():
        mean = s1_scr[...] * (1.0 / N)
        var = s2_scr[...] * (1.0 / N)
        rstd = lax.rsqrt(var + 1e-5)
        h = (z_scr[t - 2 * NB] - mean) * rstd * g_ref[0] + b_ref[0]
        out_ref[...] = jnp.maximum(h, 0.0)


def _layer_tc(agg2, h2, cnt2, wl, wr, bias2, g2, b2):
    half_a = pl.BlockSpec((BN, H), lambda h, t: (jnp.minimum(t, NB - 1), 0))
    half_b = pl.BlockSpec(
        (BN, H), lambda h, t: (NB + jnp.minimum(t, NB - 1), 0)
    )
    stat = pl.BlockSpec((1, 1, H), lambda h, t: (h, 0, 0))
    return pl.pallas_call(
        _layer_body,
        grid=(2, 3 * NB),
        in_specs=[
            half_a,
            half_b,
            half_a,
            half_b,
            half_a,
            half_b,
            pl.BlockSpec((D, H), lambda h, t: (0, h)),
            pl.BlockSpec((D, H), lambda h, t: (0, h)),
            stat,
            stat,
            stat,
        ],
        out_specs=pl.BlockSpec(
            (BN, H), lambda h, t: (h * NB + jnp.maximum(t - 2 * NB, 0), 0)
        ),
        out_shape=jax.ShapeDtypeStruct((2 * N, H), F32),
        scratch_shapes=[
            pltpu.VMEM((NB, BN, H), F32),
            pltpu.VMEM((1, H), F32),
            pltpu.VMEM((1, H), F32),
        ],
    )(agg2, agg2, h2, h2, cnt2, cnt2, wl, wr, bias2, g2, b2)


def _pool_body(batch_ref, h0_ref, h1_ref, h2_ref, h3_ref, out_ref):
    i = pl.program_id(1)
    nr = h0_ref[...] + h1_ref[...] + h2_ref[...] + h3_ref[...]
    bt = batch_ref[...]
    oh = (lax.broadcasted_iota(jnp.int32, (BN, G), 1) == bt).astype(F32)
    contrib = lax.dot_general(
        oh, nr, (((0,), (0,)), ((), ())), preferred_element_type=F32, precision=lax.Precision.HIGHEST
    )

    @pl.when(i == 0)
    def _():
        out_ref[...] = contrib

    @pl.when(i > 0)
    def _():
        out_ref[...] += contrib


def _pool(batch_col, h0, h1, h2, h3):
    blk = pl.BlockSpec((BN, H), lambda h, i: (h * NB + i, 0))
    return pl.pallas_call(
        _pool_body,
        grid=(2, NB),
        in_specs=[pl.BlockSpec((BN, 1), lambda h, i: (i, 0)), blk, blk, blk, blk],
        out_specs=pl.BlockSpec((G, H), lambda h, i: (0, h)),
        out_shape=jax.ShapeDtypeStruct((G, D), F32),
    )(batch_col, h0, h1, h2, h3)


def _mlp_body(
    hg_ref, w1_ref, b1_ref, g1_ref, bb1_ref, w2_ref, b2_ref, g2_ref, bb2_ref,
    w3_ref, b3_ref, out_ref
):
    def bn(a, g, b):
        mean = jnp.mean(a, axis=0, keepdims=True)
        d = a - mean
        var = jnp.mean(d * d, axis=0, keepdims=True)
        return d * lax.rsqrt(var + 1e-5) * g + b

    a = jnp.dot(hg_ref[...], w1_ref[...], preferred_element_type=F32) + b1_ref[...]
    a = jnp.maximum(bn(a, g1_ref[...], bb1_ref[...]), 0.0)
    a = jnp.dot(a, w2_ref[...], preferred_element_type=F32) + b2_ref[...]
    a = jnp.maximum(bn(a, g2_ref[...], bb2_ref[...]), 0.0)
    out_ref[...] = (
        jnp.dot(a, w3_ref[...], preferred_element_type=F32) + b3_ref[...]
    )


def _mlp(hg, w1, b1, g1, bb1, w2, b2, g2, bb2, w3, b3):
    return pl.pallas_call(
        _mlp_body,
        out_shape=jax.ShapeDtypeStruct((G, 1), F32),
    )(hg, w1, b1, g1, bb1, w2, b2, g2, bb2, w3, b3)


def kernel(
    x, edge_index, edge_attr, batch, node_tables, edge_tables, W_l, b_l, W_r,
    b_r, bn_g, bn_b, W1, b1, gn1_g, gn1_b, W2, b2, gn2_g, gn2_b, W3, b3
):
    src = edge_index[0]
    dst = edge_index[1]
    src2 = jnp.concatenate([src, src + N], axis=0)
    dst3 = jnp.pad(
        dst.reshape(16, _NCH, _CH), ((0, 0), (0, _NCHP - _NCH), (0, 0))
    ).reshape(16 * _NCHP, _CH)
    dstc = dst.reshape(E // _CHC, _CHC)
    tables = node_tables.reshape(9 * 128, D)

    h2 = _encode(x, tables)
    cnt2 = _count_sc(dstc)

    hs = [h2]
    for l in range(3):
        agg2 = _spmm_sc(h2, src2, dst3)
        bias2 = (b_l[l] + b_r[l]).reshape(2, 1, H)
        h2 = _layer_tc(
            agg2, h2, cnt2, W_l[l], W_r[l], bias2,
            bn_g[l].reshape(2, 1, H), bn_b[l].reshape(2, 1, H),
        )
        hs.append(h2)

    hg = _pool(batch.reshape(N, 1), hs[0], hs[1], hs[2], hs[3])
    return _mlp(
        hg,
        W1, b1.reshape(1, 2 * D), gn1_g.reshape(1, 2 * D), gn1_b.reshape(1, 2 * D),
        W2, b2.reshape(1, D), gn2_g.reshape(1, D), gn2_b.reshape(1, D),
        W3, b3.reshape(1, 1),
    )


# speedup vs baseline: 1.1511x; 1.1511x over previous
import functools

import jax
import jax.numpy as jnp
from jax import lax
from jax.experimental import pallas as pl
from jax.experimental.pallas import tpu as pltpu
from jax.experimental.pallas import tpu_sc as plsc

N = 10000
E = 160000
D = 256
G = 512
H = 128
BN = 1000
NB = N // BN
F32 = jnp.float32


_EPT = E // 16
_CH = 80
_NCH = _EPT // _CH
_NCHP = 128
_R0 = 640
_WB = 80

_EPT_C = E // 32
_CHC = 125
_NCHC = _EPT_C // _CHC


def _tile_rows(s):
    r0 = s * _R0
    nch = jnp.minimum((N - r0) // _WB, _R0 // _WB)
    return r0, nch


def _zero_acc(s, wbuf, acc):
    r0, nch = _tile_rows(s)

    def zacc(j, carry):
        pltpu.sync_copy(wbuf, acc.at[pl.ds(r0 + j * _WB, _WB)])
        return carry

    lax.fori_loop(0, nch, zacc, 0)


def _write_acc(c, s, wbuf, acc, out_hbm):
    r0, nch = _tile_rows(s)

    def wb(j, carry):
        rj = r0 + j * _WB
        pltpu.sync_copy(acc.at[pl.ds(rj, _WB)], wbuf)
        pltpu.sync_copy(wbuf, out_hbm.at[pl.ds(c * N + rj, _WB)])
        return carry

    lax.fori_loop(0, nch, wb, 0)

def _zero_vmem_rows(buf, nrows):
    zero16 = jnp.zeros((16,), F32)

    def body(r, carry):
        for k in range(H // 16):
            buf[r, pl.ds(k * 16, 16)] = zero16
        return carry

    lax.fori_loop(0, nrows, body, 0)


@functools.cache
def _make_spmm_sc():
    mesh = plsc.VectorSubcoreMesh(
        core_axis_name="c", subcore_axis_name="s", num_cores=2, num_subcores=16
    )
    return pl.kernel(
        _spmm_sc_body,
        out_type=jax.ShapeDtypeStruct((2 * N, H), F32),
        mesh=mesh,
        scratch_types=[
            pltpu.VMEM((_EPT,), jnp.int32),
            pltpu.VMEM((_NCHP, _CH), jnp.int32),
            pltpu.VMEM((_CH, H), F32),
            pltpu.VMEM((_CH, H), F32),
            pltpu.VMEM_SHARED((N, H), F32),
            pltpu.SemaphoreType.DMA,
            pltpu.SemaphoreType.DMA,
        ],
    )


def _spmm_sc(h2, src2, dst):
    return _make_spmm_sc()(h2, src2, dst)


def _spmm_sc_body(
    h_hbm, src2_hbm, dst3_hbm, out_hbm,
    isrc, idst, rowsa, rowsb, acc, sema, semb
):
    c = lax.axis_index("c")
    s = lax.axis_index("s")

    _zero_vmem_rows(rowsa, _CH)
    _zero_acc(s, rowsa, acc)
    pltpu.sync_copy(src2_hbm.at[pl.ds(c * E + s * _EPT, _EPT)], isrc)
    pltpu.sync_copy(dst3_hbm.at[pl.ds(s * _NCHP, _NCHP)], idst)
    plsc.subcore_barrier()

    def src_at(ci):
        return isrc.at[pl.ds(ci * _CH, _CH)]

    pltpu.async_copy(h_hbm.at[src_at(0)], rowsa, sema)
    bufs = ((rowsa, sema), (rowsb, semb))

    def pair(k, carry):
        for b in range(2):
            crows, csem = bufs[b]
            nrows, nsem = bufs[1 - b]
            ci = 2 * k + b
            pltpu.async_copy(h_hbm.at[src_at(ci + 1)], nrows, nsem)
            pltpu.make_async_copy(h_hbm.at[src_at(ci)], crows, csem).wait()
            pltpu.sync_copy(crows, acc.at[idst.at[ci]], add=True)
        return carry

    lax.fori_loop(0, (_NCH - 1) // 2, pair, 0)
    pltpu.make_async_copy(h_hbm.at[src_at(_NCH - 1)], rowsa, sema).wait()
    pltpu.sync_copy(rowsa, acc.at[idst.at[_NCH - 1]], add=True)
    plsc.subcore_barrier()

    _write_acc(c, s, rowsa, acc, out_hbm)


@functools.cache
def _make_count_sc():
    mesh = plsc.VectorSubcoreMesh(
        core_axis_name="c", subcore_axis_name="s", num_cores=2, num_subcores=16
    )
    return pl.kernel(
        _count_sc_body,
        out_type=jax.ShapeDtypeStruct((2 * N, H), F32),
        mesh=mesh,
        scratch_types=[
            pltpu.VMEM((_NCHC, _CHC), jnp.int32),
            pltpu.VMEM((_CHC, H), F32),
            pltpu.VMEM((_WB, H), F32),
            pltpu.VMEM_SHARED((N, H), F32),
        ],
    )


def _count_sc(dst):
    return _make_count_sc()(dst)


def _count_sc_body(dstc_hbm, out_hbm, idst, ones, wbuf, acc):
    c = lax.axis_index("c")
    s = lax.axis_index("s")

    _zero_vmem_rows(wbuf, _WB)
    one16 = jnp.ones((16,), F32)

    def fill(r, carry):
        for k in range(H // 16):
            ones[r, pl.ds(k * 16, 16)] = one16
        return carry

    lax.fori_loop(0, _CHC, fill, 0)

    _zero_acc(s, wbuf, acc)
    pltpu.sync_copy(dstc_hbm.at[pl.ds((c * 16 + s) * _NCHC, _NCHC)], idst)
    plsc.subcore_barrier()

    def step(i, carry):
        pltpu.sync_copy(ones, acc.at[idst.at[i]], add=True)
        return carry

    lax.fori_loop(0, _NCHC, step, 0)
    plsc.subcore_barrier()

    _write_acc(c, s, wbuf, acc, out_hbm)


def _encode_body(x_ref, t_ref, out_ref):
    x = x_ref[...]
    acc = jnp.zeros((BN, H), F32)
    tb = t_ref[...]
    for f in range(9):
        col = x[:, f : f + 1]
        oh = (lax.broadcasted_iota(jnp.int32, (BN, 128), 1) == col).astype(F32)
        acc = acc + jnp.dot(
            oh, tb[f * 128 : (f + 1) * 128, :], preferred_element_type=F32, precision=lax.Precision.HIGHEST
        )
    out_ref[...] = acc


def _encode(x, tables):
    return pl.pallas_call(
        _encode_body,
        grid=(2, NB),
        in_specs=[
            pl.BlockSpec((BN, 9), lambda h, i: (i, 0)),
            pl.BlockSpec((9 * 128, H), lambda h, i: (0, h)),
        ],
        out_specs=pl.BlockSpec((BN, H), lambda h, i: (h * NB + i, 0)),
        out_shape=jax.ShapeDtypeStruct((2 * N, H), F32),
    )(x, tables)


def _layer_body(
    agg_a, agg_b, h_a, h_b, cnt_a, cnt_b, wl, wr, bias, g_ref, b_ref,
    out_ref, z_scr, s1_scr, s2_scr
):
    t = pl.program_id(1)

    @pl.when(t < NB)
    def _():
        inv = 1.0 / jnp.maximum(cnt_a[...] + cnt_b[...], 1.0)
        m = jnp.concatenate([agg_a[...] * inv, agg_b[...] * inv], axis=1)
        hp = jnp.concatenate([h_a[...], h_b[...]], axis=1)
        z = (
            jnp.dot(m, wl[...], preferred_element_type=F32)
            + jnp.dot(hp, wr[...], preferred_element_type=F32)
            + bias[0]
        )
        z_scr[t] = z
        cs = jnp.sum(z, axis=0, keepdims=True)

        @pl.when(t == 0)
        def _():
            s1_scr[...] = cs

        @pl.when(t > 0)
        def _():
            s1_scr[...] += cs

    @pl.when((t >= NB) & (t < 2 * NB))
    def _():
        mean = s1_scr[...] * (1.0 / N)
        dz = z_scr[t - NB] - mean
        cv = jnp.sum(dz * dz, axis=0, keepdims=True)

        @pl.when(t == NB)
        def _():
            s2_scr[...] = cv

        @pl.when(t > NB)
        def _():
            s2_scr[...] += cv

    @pl.when(t >= 2 * NB)
    def _():
        mean = s1_scr[...] * (1.0 / N)
        var = s2_scr[...] * (1.0 / N)
        rstd = lax.rsqrt(var + 1e-5)
        h = (z_scr[t - 2 * NB] - mean) * rstd * g_ref[0] + b_ref[0]
        out_ref[...] = jnp.maximum(h, 0.0)


def _layer_tc(agg2, h2, cnt2, wl, wr, bias2, g2, b2):
    half_a = pl.BlockSpec((BN, H), lambda h, t: (jnp.minimum(t, NB - 1), 0))
    half_b = pl.BlockSpec(
        (BN, H), lambda h, t: (NB + jnp.minimum(t, NB - 1), 0)
    )
    stat = pl.BlockSpec((1, 1, H), lambda h, t: (h, 0, 0))
    return pl.pallas_call(
        _layer_body,
        grid=(2, 3 * NB),
        in_specs=[
            half_a,
            half_b,
            half_a,
            half_b,
            half_a,
            half_b,
            pl.BlockSpec((D, H), lambda h, t: (0, h)),
            pl.BlockSpec((D, H), lambda h, t: (0, h)),
            stat,
            stat,
            stat,
        ],
        out_specs=pl.BlockSpec(
            (BN, H), lambda h, t: (h * NB + jnp.maximum(t - 2 * NB, 0), 0)
        ),
        out_shape=jax.ShapeDtypeStruct((2 * N, H), F32),
        scratch_shapes=[
            pltpu.VMEM((NB, BN, H), F32),
            pltpu.VMEM((1, H), F32),
            pltpu.VMEM((1, H), F32),
        ],
    )(agg2, agg2, h2, h2, cnt2, cnt2, wl, wr, bias2, g2, b2)


def _pool_body(batch_ref, h0_ref, h1_ref, h2_ref, h3_ref, out_ref):
    i = pl.program_id(1)
    nr = h0_ref[...] + h1_ref[...] + h2_ref[...] + h3_ref[...]
    bt = batch_ref[...]
    oh = (lax.broadcasted_iota(jnp.int32, (BN, G), 1) == bt).astype(F32)
    contrib = lax.dot_general(
        oh, nr, (((0,), (0,)), ((), ())), preferred_element_type=F32, precision=lax.Precision.HIGHEST
    )

    @pl.when(i == 0)
    def _():
        out_ref[...] = contrib

    @pl.when(i > 0)
    def _():
        out_ref[...] += contrib


def _pool(batch_col, h0, h1, h2, h3):
    blk = pl.BlockSpec((BN, H), lambda h, i: (h * NB + i, 0))
    return pl.pallas_call(
        _pool_body,
        grid=(2, NB),
        in_specs=[pl.BlockSpec((BN, 1), lambda h, i: (i, 0)), blk, blk, blk, blk],
        out_specs=pl.BlockSpec((G, H), lambda h, i: (0, h)),
        out_shape=jax.ShapeDtypeStruct((G, D), F32),
    )(batch_col, h0, h1, h2, h3)


def _mlp_body(
    hg_ref, w1_ref, b1_ref, g1_ref, bb1_ref, w2_ref, b2_ref, g2_ref, bb2_ref,
    w3_ref, b3_ref, out_ref
):
    def bn(a, g, b):
        mean = jnp.mean(a, axis=0, keepdims=True)
        d = a - mean
        var = jnp.mean(d * d, axis=0, keepdims=True)
        return d * lax.rsqrt(var + 1e-5) * g + b

    a = jnp.dot(hg_ref[...], w1_ref[...], preferred_element_type=F32) + b1_ref[...]
    a = jnp.maximum(bn(a, g1_ref[...], bb1_ref[...]), 0.0)
    a = jnp.dot(a, w2_ref[...], preferred_element_type=F32) + b2_ref[...]
    a = jnp.maximum(bn(a, g2_ref[...], bb2_ref[...]), 0.0)
    out_ref[...] = (
        jnp.dot(a, w3_ref[...], preferred_element_type=F32) + b3_ref[...]
    )


def _mlp(hg, w1, b1, g1, bb1, w2, b2, g2, bb2, w3, b3):
    return pl.pallas_call(
        _mlp_body,
        out_shape=jax.ShapeDtypeStruct((G, 1), F32),
    )(hg, w1, b1, g1, bb1, w2, b2, g2, bb2, w3, b3)


def kernel(
    x, edge_index, edge_attr, batch, node_tables, edge_tables, W_l, b_l, W_r,
    b_r, bn_g, bn_b, W1, b1, gn1_g, gn1_b, W2, b2, gn2_g, gn2_b, W3, b3
):
    src = edge_index[0]
    dst = edge_index[1]
    src2 = jnp.concatenate([src, src + N], axis=0)
    dst3 = jnp.pad(
        dst.reshape(16, _NCH, _CH), ((0, 0), (0, _NCHP - _NCH), (0, 0))
    ).reshape(16 * _NCHP, _CH)
    dstc = dst.reshape(E // _CHC, _CHC)
    tables = node_tables.reshape(9 * 128, D)

    h2 = _encode(x, tables)
    cnt2 = _count_sc(dstc)

    hs = [h2]
    for l in range(3):
        agg2 = _spmm_sc(h2, src2, dst3)
        bias2 = (b_l[l] + b_r[l]).reshape(2, 1, H)
        h2 = _layer_tc(
            agg2, h2, cnt2, W_l[l], W_r[l], bias2,
            bn_g[l].reshape(2, 1, H), bn_b[l].reshape(2, 1, H),
        )
        hs.append(h2)

    hg = _pool(batch.reshape(N, 1), hs[0], hs[1], hs[2], hs[3])
    return _mlp(
        hg,
        W1, b1.reshape(1, 2 * D), gn1_g.reshape(1, 2 * D), gn1_b.reshape(1, 2 * D),
        W2, b2.reshape(1, D), gn2_g.reshape(1, D), gn2_b.reshape(1, D),
        W3, b3.reshape(1, 1),
    )


# speedup vs baseline: 1.1840x; 1.0286x over previous
import functools

import jax
import jax.numpy as jnp
from jax import lax
from jax.experimental import pallas as pl
from jax.experimental.pallas import tpu as pltpu
from jax.experimental.pallas import tpu_sc as plsc

N = 10000
E = 160000
D = 256
G = 512
H = 128
HC = 16
BN = 1000
NB = N // BN
F32 = jnp.float32


_EPT = E // 16
_CH = 80
_NCH = _EPT // _CH
_NCHP = 128
_R0 = 640
_WB = 80

_EPT_C = E // 32
_CHC = 125
_NCHC = _EPT_C // _CHC


def _tile_rows(s):
    r0 = s * _R0
    nch = jnp.minimum((N - r0) // _WB, _R0 // _WB)
    return r0, nch


def _zero_acc(s, wbuf, acc):
    r0, nch = _tile_rows(s)

    def zacc(j, carry):
        pltpu.sync_copy(wbuf, acc.at[pl.ds(r0 + j * _WB, _WB)])
        return carry

    lax.fori_loop(0, nch, zacc, 0)


def _write_acc(c, s, wbuf, acc, out_hbm):
    r0, nch = _tile_rows(s)

    def wb(j, carry):
        rj = r0 + j * _WB
        pltpu.sync_copy(acc.at[pl.ds(rj, _WB)], wbuf)
        pltpu.sync_copy(wbuf, out_hbm.at[pl.ds(c * N + rj, _WB)])
        return carry

    lax.fori_loop(0, nch, wb, 0)

def _zero_vmem_rows(buf, nrows):
    zero16 = jnp.zeros((16,), F32)

    def body(r, carry):
        for k in range(H // 16):
            buf[r, pl.ds(k * 16, 16)] = zero16
        return carry

    lax.fori_loop(0, nrows, body, 0)


@functools.cache
def _make_spmm_sc():
    mesh = plsc.VectorSubcoreMesh(
        core_axis_name="c", subcore_axis_name="s", num_cores=2, num_subcores=16
    )
    return pl.kernel(
        _spmm_sc_body,
        out_type=jax.ShapeDtypeStruct((2 * N, H), F32),
        mesh=mesh,
        scratch_types=[
            pltpu.VMEM((_EPT,), jnp.int32),
            pltpu.VMEM((_NCHP, _CH), jnp.int32),
            pltpu.VMEM((_CH, H), F32),
            pltpu.VMEM((_CH, H), F32),
            pltpu.VMEM_SHARED((N, H), F32),
            pltpu.SemaphoreType.DMA,
            pltpu.SemaphoreType.DMA,
        ],
    )


def _spmm_sc(h2, src2, dst):
    return _make_spmm_sc()(h2, src2, dst)


def _spmm_sc_body(
    h_hbm, src2_hbm, dst3_hbm, out_hbm,
    isrc, idst, rowsa, rowsb, acc, sema, semb
):
    c = lax.axis_index("c")
    s = lax.axis_index("s")

    _zero_vmem_rows(rowsa, _CH)
    _zero_acc(s, rowsa, acc)
    pltpu.sync_copy(src2_hbm.at[pl.ds(c * E + s * _EPT, _EPT)], isrc)
    pltpu.sync_copy(dst3_hbm.at[pl.ds(s * _NCHP, _NCHP)], idst)
    plsc.subcore_barrier()

    def src_at(ci):
        return isrc.at[pl.ds(ci * _CH, _CH)]

    pltpu.async_copy(h_hbm.at[src_at(0)], rowsa, sema)
    bufs = ((rowsa, sema), (rowsb, semb))

    def pair(k, carry):
        for b in range(2):
            crows, csem = bufs[b]
            nrows, nsem = bufs[1 - b]
            ci = 2 * k + b
            pltpu.async_copy(h_hbm.at[src_at(ci + 1)], nrows, nsem)
            pltpu.make_async_copy(h_hbm.at[src_at(ci)], crows, csem).wait()
            pltpu.sync_copy(crows, acc.at[idst.at[ci]], add=True)
        return carry

    lax.fori_loop(0, (_NCH - 1) // 2, pair, 0)
    pltpu.make_async_copy(h_hbm.at[src_at(_NCH - 1)], rowsa, sema).wait()
    pltpu.sync_copy(rowsa, acc.at[idst.at[_NCH - 1]], add=True)
    plsc.subcore_barrier()

    _write_acc(c, s, rowsa, acc, out_hbm)


@functools.cache
def _make_count_sc():
    mesh = plsc.VectorSubcoreMesh(
        core_axis_name="c", subcore_axis_name="s", num_cores=2, num_subcores=16
    )
    return pl.kernel(
        _count_sc_body,
        out_type=jax.ShapeDtypeStruct((2 * N, HC), F32),
        mesh=mesh,
        scratch_types=[
            pltpu.VMEM((_NCHC, _CHC), jnp.int32),
            pltpu.VMEM((_CHC, HC), F32),
            pltpu.VMEM((_WB, HC), F32),
            pltpu.VMEM_SHARED((N, HC), F32),
        ],
    )


def _count_sc(dst):
    return _make_count_sc()(dst)


def _count_sc_body(dstc_hbm, out_hbm, idst, ones, wbuf, acc):
    c = lax.axis_index("c")
    s = lax.axis_index("s")

    zero16 = jnp.zeros((16,), F32)
    one16 = jnp.ones((16,), F32)

    def fill(r, carry):
        ones[r, pl.ds(0, HC)] = one16
        wbuf[r, pl.ds(0, HC)] = zero16
        return carry

    lax.fori_loop(0, _CHC, fill, 0)
    r0, nch = _tile_rows(s)

    def zacc(j, carry):
        pltpu.sync_copy(wbuf.at[pl.ds(0, _WB)], acc.at[pl.ds(r0 + j * _WB, _WB)])
        return carry

    lax.fori_loop(0, nch, zacc, 0)
    pltpu.sync_copy(dstc_hbm.at[pl.ds((c * 16 + s) * _NCHC, _NCHC)], idst)
    plsc.subcore_barrier()

    def step(i, carry):
        pltpu.sync_copy(ones, acc.at[idst.at[i]], add=True)
        return carry

    lax.fori_loop(0, _NCHC, step, 0)
    plsc.subcore_barrier()

    def wb2(j, carry):
        rj = r0 + j * _WB
        pltpu.sync_copy(acc.at[pl.ds(rj, _WB)], wbuf.at[pl.ds(0, _WB)])
        pltpu.sync_copy(wbuf.at[pl.ds(0, _WB)], out_hbm.at[pl.ds(c * N + rj, _WB)])
        return carry

    lax.fori_loop(0, nch, wb2, 0)


def _encode_body(x_ref, t_ref, out_ref):
    x = x_ref[...]
    acc = jnp.zeros((BN, H), F32)
    tb = t_ref[...]
    for f in range(9):
        col = x[:, f : f + 1]
        oh = (lax.broadcasted_iota(jnp.int32, (BN, 128), 1) == col).astype(F32)
        acc = acc + jnp.dot(
            oh, tb[f * 128 : (f + 1) * 128, :], preferred_element_type=F32, precision=lax.Precision.HIGHEST
        )
    out_ref[...] = acc


def _encode(x, tables):
    return pl.pallas_call(
        _encode_body,
        grid=(2, NB),
        in_specs=[
            pl.BlockSpec((BN, 9), lambda h, i: (i, 0)),
            pl.BlockSpec((9 * 128, H), lambda h, i: (0, h)),
        ],
        out_specs=pl.BlockSpec((BN, H), lambda h, i: (h * NB + i, 0)),
        out_shape=jax.ShapeDtypeStruct((2 * N, H), F32),
    )(x, tables)


def _layer_body(
    agg_a, agg_b, h_a, h_b, cnt_a, cnt_b, wl, wr, bias, g_ref, b_ref,
    out_ref, z_scr, s1_scr, s2_scr
):
    t = pl.program_id(1)

    @pl.when(t < NB)
    def _():
        inv = 1.0 / jnp.maximum(cnt_a[...][:, :1] + cnt_b[...][:, :1], 1.0)
        m = jnp.concatenate([agg_a[...] * inv, agg_b[...] * inv], axis=1)
        hp = jnp.concatenate([h_a[...], h_b[...]], axis=1)
        z = (
            jnp.dot(m, wl[...], preferred_element_type=F32)
            + jnp.dot(hp, wr[...], preferred_element_type=F32)
            + bias[0]
        )
        z_scr[t] = z
        cs = jnp.sum(z, axis=0, keepdims=True)

        @pl.when(t == 0)
        def _():
            s1_scr[...] = cs

        @pl.when(t > 0)
        def _():
            s1_scr[...] += cs

    @pl.when((t >= NB) & (t < 2 * NB))
    def _():
        mean = s1_scr[...] * (1.0 / N)
        dz = z_scr[t - NB] - mean
        cv = jnp.sum(dz * dz, axis=0, keepdims=True)

        @pl.when(t == NB)
        def _():
            s2_scr[...] = cv

        @pl.when(t > NB)
        def _():
            s2_scr[...] += cv

    @pl.when(t >= 2 * NB)
    def _():
        mean = s1_scr[...] * (1.0 / N)
        var = s2_scr[...] * (1.0 / N)
        rstd = lax.rsqrt(var + 1e-5)
        h = (z_scr[t - 2 * NB] - mean) * rstd * g_ref[0] + b_ref[0]
        out_ref[...] = jnp.maximum(h, 0.0)


def _layer_tc(agg2, h2, cnt2, wl, wr, bias2, g2, b2):
    half_a = pl.BlockSpec((BN, H), lambda h, t: (jnp.minimum(t, NB - 1), 0))
    half_b = pl.BlockSpec(
        (BN, H), lambda h, t: (NB + jnp.minimum(t, NB - 1), 0)
    )
    cnt_a = pl.BlockSpec((BN, HC), lambda h, t: (jnp.minimum(t, NB - 1), 0))
    cnt_b = pl.BlockSpec(
        (BN, HC), lambda h, t: (NB + jnp.minimum(t, NB - 1), 0)
    )
    stat = pl.BlockSpec((1, 1, H), lambda h, t: (h, 0, 0))
    return pl.pallas_call(
        _layer_body,
        grid=(2, 3 * NB),
        in_specs=[
            half_a,
            half_b,
            half_a,
            half_b,
            cnt_a,
            cnt_b,
            pl.BlockSpec((D, H), lambda h, t: (0, h)),
            pl.BlockSpec((D, H), lambda h, t: (0, h)),
            stat,
            stat,
            stat,
        ],
        out_specs=pl.BlockSpec(
            (BN, H), lambda h, t: (h * NB + jnp.maximum(t - 2 * NB, 0), 0)
        ),
        out_shape=jax.ShapeDtypeStruct((2 * N, H), F32),
        scratch_shapes=[
            pltpu.VMEM((NB, BN, H), F32),
            pltpu.VMEM((1, H), F32),
            pltpu.VMEM((1, H), F32),
        ],
    )(agg2, agg2, h2, h2, cnt2, cnt2, wl, wr, bias2, g2, b2)


def _pool_body(batch_ref, h0_ref, h1_ref, h2_ref, h3_ref, out_ref):
    i = pl.program_id(1)
    nr = h0_ref[...] + h1_ref[...] + h2_ref[...] + h3_ref[...]
    bt = batch_ref[...]
    oh = (lax.broadcasted_iota(jnp.int32, (BN, G), 1) == bt).astype(F32)
    contrib = lax.dot_general(
        oh, nr, (((0,), (0,)), ((), ())), preferred_element_type=F32, precision=lax.Precision.HIGHEST
    )

    @pl.when(i == 0)
    def _():
        out_ref[...] = contrib

    @pl.when(i > 0)
    def _():
        out_ref[...] += contrib


def _pool(batch_col, h0, h1, h2, h3):
    blk = pl.BlockSpec((BN, H), lambda h, i: (h * NB + i, 0))
    return pl.pallas_call(
        _pool_body,
        grid=(2, NB),
        in_specs=[pl.BlockSpec((BN, 1), lambda h, i: (i, 0)), blk, blk, blk, blk],
        out_specs=pl.BlockSpec((G, H), lambda h, i: (0, h)),
        out_shape=jax.ShapeDtypeStruct((G, D), F32),
    )(batch_col, h0, h1, h2, h3)


def _mlp_body(
    hg_ref, w1_ref, b1_ref, g1_ref, bb1_ref, w2_ref, b2_ref, g2_ref, bb2_ref,
    w3_ref, b3_ref, out_ref
):
    def bn(a, g, b):
        mean = jnp.mean(a, axis=0, keepdims=True)
        d = a - mean
        var = jnp.mean(d * d, axis=0, keepdims=True)
        return d * lax.rsqrt(var + 1e-5) * g + b

    a = jnp.dot(hg_ref[...], w1_ref[...], preferred_element_type=F32) + b1_ref[...]
    a = jnp.maximum(bn(a, g1_ref[...], bb1_ref[...]), 0.0)
    a = jnp.dot(a, w2_ref[...], preferred_element_type=F32) + b2_ref[...]
    a = jnp.maximum(bn(a, g2_ref[...], bb2_ref[...]), 0.0)
    out_ref[...] = (
        jnp.dot(a, w3_ref[...], preferred_element_type=F32) + b3_ref[...]
    )


def _mlp(hg, w1, b1, g1, bb1, w2, b2, g2, bb2, w3, b3):
    return pl.pallas_call(
        _mlp_body,
        out_shape=jax.ShapeDtypeStruct((G, 1), F32),
    )(hg, w1, b1, g1, bb1, w2, b2, g2, bb2, w3, b3)


def kernel(
    x, edge_index, edge_attr, batch, node_tables, edge_tables, W_l, b_l, W_r,
    b_r, bn_g, bn_b, W1, b1, gn1_g, gn1_b, W2, b2, gn2_g, gn2_b, W3, b3
):
    src = edge_index[0]
    dst = edge_index[1]
    src2 = jnp.concatenate([src, src + N], axis=0)
    dst3 = jnp.pad(
        dst.reshape(16, _NCH, _CH), ((0, 0), (0, _NCHP - _NCH), (0, 0))
    ).reshape(16 * _NCHP, _CH)
    dstc = dst.reshape(E // _CHC, _CHC)
    tables = node_tables.reshape(9 * 128, D)

    h2 = _encode(x, tables)
    cnt2 = _count_sc(dstc)

    hs = [h2]
    for l in range(3):
        agg2 = _spmm_sc(h2, src2, dst3)
        bias2 = (b_l[l] + b_r[l]).reshape(2, 1, H)
        h2 = _layer_tc(
            agg2, h2, cnt2, W_l[l], W_r[l], bias2,
            bn_g[l].reshape(2, 1, H), bn_b[l].reshape(2, 1, H),
        )
        hs.append(h2)

    hg = _pool(batch.reshape(N, 1), hs[0], hs[1], hs[2], hs[3])
    return _mlp(
        hg,
        W1, b1.reshape(1, 2 * D), gn1_g.reshape(1, 2 * D), gn1_b.reshape(1, 2 * D),
        W2, b2.reshape(1, D), gn2_g.reshape(1, D), gn2_b.reshape(1, D),
        W3, b3.reshape(1, 1),
    )
